# Initial kernel scaffold; baseline (speedup 1.0000x reference)
import functools

import jax
import jax.numpy as jnp
from jax import lax
from jax.experimental import pallas as pl
from jax.experimental.pallas import tpu as pltpu
from jax.experimental.pallas import tpu_sc as plsc

_ALPHA = 0.1
_K = 10
_CHUNK = 128

_MESH = plsc.VectorSubcoreMesh(core_axis_name="c", subcore_axis_name="s")
_NC = 2
_NS = 16


def _deg_body(n, e, dst_hbm, zeros_hbm, ones_hbm, out_hbm,
              acc, ones_v, ones_t, idx_b, idx_t):
    c = lax.axis_index("c")
    s = lax.axis_index("s")
    rows = n // _NS
    per_tile = e // (_NC * _NS)
    n_chunks = per_tile // _CHUNK
    tail = per_tile - n_chunks * _CHUNK
    base = (c * _NS + s) * per_tile

    pltpu.sync_copy(zeros_hbm.at[pl.ds(s * rows, rows)],
                    acc.at[pl.ds(s * rows, rows)])
    pltpu.sync_copy(ones_hbm, ones_v)
    if tail:
        pltpu.sync_copy(ones_hbm.at[pl.ds(0, tail)], ones_t)
    plsc.subcore_barrier()

    def body(i, carry):
        off = pl.multiple_of(base + i * _CHUNK, 8)
        pltpu.sync_copy(dst_hbm.at[pl.ds(off, _CHUNK)], idx_b)
        pltpu.sync_copy(ones_v, acc.at[idx_b], add=True)
        return carry

    lax.fori_loop(0, n_chunks, body, 0)
    if tail:
        off = pl.multiple_of(base + n_chunks * _CHUNK, 8)
        pltpu.sync_copy(dst_hbm.at[pl.ds(off, tail)], idx_t)
        pltpu.sync_copy(ones_t, acc.at[idx_t], add=True)
    plsc.subcore_barrier()
    pltpu.sync_copy(acc.at[pl.ds(s * rows, rows)],
                    out_hbm.at[c, pl.ds(s * rows, rows)])


def _sc_degree(dst, n):
    e = dst.shape[0]
    per_tile = e // (_NC * _NS)
    tail = per_tile % _CHUNK
    zeros = jnp.zeros((n, 8), jnp.float32)
    ones = jnp.ones((_CHUNK, 8), jnp.float32)
    scratch = [
        pltpu.VMEM_SHARED((n, 8), jnp.float32),
        pltpu.VMEM((_CHUNK, 8), jnp.float32),
        pltpu.VMEM((max(tail, 8), 8), jnp.float32),
        pltpu.VMEM((_CHUNK,), jnp.int32),
        pltpu.VMEM((max(tail, 8),), jnp.int32),
    ]
    kfn = pl.kernel(
        functools.partial(_deg_body, n, e),
        out_type=jax.ShapeDtypeStruct((_NC, n, 8), jnp.float32),
        mesh=_MESH,
        scratch_types=scratch,
    )
    return kfn(dst, zeros, ones)


def _scat_body(n, e, d, u_hbm, src_hbm, dst_hbm, zeros_hbm, out_hbm,
               acc, src_b, dst_b, src_t, dst_t, rows_b, rows_t, sem):
    c = lax.axis_index("c")
    s = lax.axis_index("s")
    rows = n // _NS
    per_tile = e // (_NC * _NS)
    n_chunks = per_tile // _CHUNK
    tail = per_tile - n_chunks * _CHUNK
    base = (c * _NS + s) * per_tile

    pltpu.sync_copy(zeros_hbm.at[pl.ds(s * rows, rows)],
                    acc.at[pl.ds(s * rows, rows)])
    plsc.subcore_barrier()

    def body(i, carry):
        off = pl.multiple_of(base + i * _CHUNK, 8)
        pltpu.sync_copy(src_hbm.at[pl.ds(off, _CHUNK)], src_b)
        pltpu.sync_copy(dst_hbm.at[pl.ds(off, _CHUNK)], dst_b)
        pltpu.async_copy(u_hbm.at[src_b], rows_b, sem).wait()
        pltpu.sync_copy(rows_b, acc.at[dst_b], add=True)
        return carry

    lax.fori_loop(0, n_chunks, body, 0)
    if tail:
        off = pl.multiple_of(base + n_chunks * _CHUNK, 8)
        pltpu.sync_copy(src_hbm.at[pl.ds(off, tail)], src_t)
        pltpu.sync_copy(dst_hbm.at[pl.ds(off, tail)], dst_t)
        pltpu.async_copy(u_hbm.at[src_t], rows_t, sem).wait()
        pltpu.sync_copy(rows_t, acc.at[dst_t], add=True)
    plsc.subcore_barrier()
    pltpu.sync_copy(acc.at[pl.ds(s * rows, rows)],
                    out_hbm.at[c, pl.ds(s * rows, rows)])


def _make_scat(n, e, d):
    per_tile = e // (_NC * _NS)
    tail = per_tile % _CHUNK
    scratch = [
        pltpu.VMEM_SHARED((n, d), jnp.float32),
        pltpu.VMEM((_CHUNK,), jnp.int32),
        pltpu.VMEM((_CHUNK,), jnp.int32),
        pltpu.VMEM((max(tail, 8),), jnp.int32),
        pltpu.VMEM((max(tail, 8),), jnp.int32),
        pltpu.VMEM((_CHUNK, d), jnp.float32),
        pltpu.VMEM((max(tail, 8), d), jnp.float32),
        pltpu.SemaphoreType.DMA,
    ]
    return pl.kernel(
        functools.partial(_scat_body, n, e, d),
        out_type=jax.ShapeDtypeStruct((_NC, n, d), jnp.float32),
        mesh=_MESH,
        scratch_types=scratch,
    )


def _prep_body(x_ref, w1_ref, b1_ref, w2_ref, b2_ref, d0_ref, d1_ref,
               u0_ref, ch_ref, c2_ref, sq_ref):
    x = x_ref[...]
    h1 = jax.nn.relu(
        lax.dot_general(x, w1_ref[...], (((1,), (1,)), ((), ())),
                        preferred_element_type=jnp.float32) + b1_ref[...])
    h2 = lax.dot_general(h1, w2_ref[...], (((1,), (1,)), ((), ())),
                         preferred_element_type=jnp.float32) + b2_ref[...]
    deg = d0_ref[...] + d1_ref[...] + 1.0
    dinv = lax.rsqrt(deg)[:, None]
    u0 = h2 * dinv
    u0_ref[...] = u0
    ch_ref[...] = _ALPHA * u0
    c2_ref[...] = jnp.broadcast_to((1.0 - _ALPHA) * dinv * dinv, h2.shape)
    sq_ref[...] = jnp.broadcast_to(jnp.sqrt(deg)[:, None], h2.shape)


def _tc_prep(x, w1, b1, w2, b2, d0, d1):
    n, fin = x.shape
    d = w2.shape[0]
    blk = 2000
    grid = (n // blk,)
    f32 = jnp.float32
    out_shape = [jax.ShapeDtypeStruct((n, d), f32)] * 4
    return pl.pallas_call(
        _prep_body,
        grid=grid,
        in_specs=[
            pl.BlockSpec((blk, fin), lambda i: (i, 0)),
            pl.BlockSpec(w1.shape, lambda i: (0, 0)),
            pl.BlockSpec((1, w1.shape[0]), lambda i: (0, 0)),
            pl.BlockSpec(w2.shape, lambda i: (0, 0)),
            pl.BlockSpec((1, w2.shape[0]), lambda i: (0, 0)),
            pl.BlockSpec((blk,), lambda i: (i,)),
            pl.BlockSpec((blk,), lambda i: (i,)),
        ],
        out_specs=[pl.BlockSpec((blk, d), lambda i: (i, 0))] * 4,
        out_shape=out_shape,
    )(x, w1, b1.reshape(1, -1), w2, b2.reshape(1, -1), d0, d1)


def _comb_body(s0_ref, s1_ref, u_ref, c2_ref, ch_ref, o_ref):
    o_ref[...] = (c2_ref[...] * (s0_ref[...] + s1_ref[...] + u_ref[...])
                  + ch_ref[...])


def _tc_combine(s0, s1, u, c2, ch):
    n, d = u.shape
    blk = 2000
    spec = pl.BlockSpec((blk, d), lambda i: (i, 0))
    return pl.pallas_call(
        _comb_body,
        grid=(n // blk,),
        in_specs=[spec] * 5,
        out_specs=spec,
        out_shape=jax.ShapeDtypeStruct((n, d), jnp.float32),
    )(s0, s1, u, c2, ch)


def _final_body(u_ref, sq_ref, o_ref):
    z = u_ref[...] * sq_ref[...]
    m = jnp.max(z, axis=1, keepdims=True)
    lse = jnp.log(jnp.sum(jnp.exp(z - m), axis=1, keepdims=True)) + m
    o_ref[...] = z - lse


def _tc_final(u, sq):
    n, d = u.shape
    blk = 2000
    spec = pl.BlockSpec((blk, d), lambda i: (i, 0))
    return pl.pallas_call(
        _final_body,
        grid=(n // blk,),
        in_specs=[spec] * 2,
        out_specs=spec,
        out_shape=jax.ShapeDtypeStruct((n, d), jnp.float32),
    )(u, sq)


def kernel(x, edge_index, W1, b1, W2, b2):
    n = x.shape[0]
    d = W2.shape[0]
    src = edge_index[0]
    dst = edge_index[1]

    degp = _sc_degree(dst, n)
    u0, ch, c2, sq = _tc_prep(x, W1, b1, W2, b2, degp[0, :, 0], degp[1, :, 0])

    scat_fn = _make_scat(n, src.shape[0], d)
    zeros = jnp.zeros((n, d), jnp.float32)
    u = u0
    for _ in range(_K):
        scat = scat_fn(u, src, dst, zeros)
        u = _tc_combine(scat[0], scat[1], u, c2, ch)
    return _tc_final(u, sq)


# speedup vs baseline: 12.3675x; 12.3675x over previous
import functools

import jax
import jax.numpy as jnp
from jax import lax
from jax.experimental import pallas as pl
from jax.experimental.pallas import tpu as pltpu
from jax.experimental.pallas import tpu_sc as plsc

_ALPHA = 0.1
_K = 10
_CHUNK = 128

_NC = 2
_NS = 16
_MESH = plsc.VectorSubcoreMesh(core_axis_name="c", subcore_axis_name="s",
                               num_cores=_NC, num_subcores=_NS)


def _rows_per_tile(n):
    return -(-n // (_NS * 8)) * 8


def _deg_body(n, e, dst_hbm, zeros_hbm, ones_hbm, out_hbm,
              acc, ones_v, ones_t, idx_b, idx_t):
    c = lax.axis_index("c")
    s = lax.axis_index("s")
    rows = _rows_per_tile(n)
    per_tile = e // (_NC * _NS)
    n_chunks = per_tile // _CHUNK
    tail = per_tile - n_chunks * _CHUNK
    base = (c * _NS + s) * per_tile

    pltpu.sync_copy(zeros_hbm.at[pl.ds(s * rows, rows)],
                    acc.at[pl.ds(s * rows, rows)])
    pltpu.sync_copy(ones_hbm, ones_v)
    if tail:
        pltpu.sync_copy(ones_hbm.at[pl.ds(0, tail)], ones_t)
    plsc.subcore_barrier()

    def body(i, carry):
        off = pl.multiple_of(base + i * _CHUNK, 8)
        pltpu.sync_copy(dst_hbm.at[pl.ds(off, _CHUNK)], idx_b)
        pltpu.sync_copy(ones_v, acc.at[idx_b], add=True)
        return carry

    lax.fori_loop(0, n_chunks, body, 0)
    if tail:
        off = pl.multiple_of(base + n_chunks * _CHUNK, 8)
        pltpu.sync_copy(dst_hbm.at[pl.ds(off, tail)], idx_t)
        pltpu.sync_copy(ones_t, acc.at[idx_t], add=True)
    plsc.subcore_barrier()
    pltpu.sync_copy(acc.at[pl.ds(s * rows, rows)],
                    out_hbm.at[c, pl.ds(s * rows, rows)])


def _sc_degree(dst, n):
    e = dst.shape[0]
    per_tile = e // (_NC * _NS)
    tail = per_tile % _CHUNK
    n_pad = _rows_per_tile(n) * _NS
    zeros = jnp.zeros((n_pad, 8), jnp.float32)
    ones = jnp.ones((_CHUNK, 8), jnp.float32)
    scratch = [
        pltpu.VMEM_SHARED((n_pad, 8), jnp.float32),
        pltpu.VMEM((_CHUNK, 8), jnp.float32),
        pltpu.VMEM((max(tail, 8), 8), jnp.float32),
        pltpu.VMEM((_CHUNK,), jnp.int32),
        pltpu.VMEM((max(tail, 8),), jnp.int32),
    ]
    kfn = pl.kernel(
        functools.partial(_deg_body, n, e),
        out_type=jax.ShapeDtypeStruct((_NC, n_pad, 8), jnp.float32),
        mesh=_MESH,
        scratch_types=scratch,
        compiler_params=pltpu.CompilerParams(use_tc_tiling_on_sc=False),
    )
    return kfn(dst, zeros, ones)


def _scat_body(n, e, d, u_hbm, src_hbm, dst_hbm, zeros_hbm, out_hbm,
               acc, src_b, dst_b, src_t, dst_t, rows_b, rows_t, sem):
    c = lax.axis_index("c")
    s = lax.axis_index("s")
    rows = _rows_per_tile(n)
    per_tile = e // (_NC * _NS)
    n_chunks = per_tile // _CHUNK
    tail = per_tile - n_chunks * _CHUNK
    base = (c * _NS + s) * per_tile

    pltpu.sync_copy(zeros_hbm.at[pl.ds(s * rows, rows)],
                    acc.at[pl.ds(s * rows, rows)])
    plsc.subcore_barrier()

    def body(i, carry):
        off = pl.multiple_of(base + i * _CHUNK, 8)
        pltpu.sync_copy(src_hbm.at[pl.ds(off, _CHUNK)], src_b)
        pltpu.sync_copy(dst_hbm.at[pl.ds(off, _CHUNK)], dst_b)
        pltpu.async_copy(u_hbm.at[src_b], rows_b, sem).wait()
        pltpu.sync_copy(rows_b, acc.at[dst_b], add=True)
        return carry

    lax.fori_loop(0, n_chunks, body, 0)
    if tail:
        off = pl.multiple_of(base + n_chunks * _CHUNK, 8)
        pltpu.sync_copy(src_hbm.at[pl.ds(off, tail)], src_t)
        pltpu.sync_copy(dst_hbm.at[pl.ds(off, tail)], dst_t)
        pltpu.async_copy(u_hbm.at[src_t], rows_t, sem).wait()
        pltpu.sync_copy(rows_t, acc.at[dst_t], add=True)
    plsc.subcore_barrier()
    pltpu.sync_copy(acc.at[pl.ds(s * rows, rows)],
                    out_hbm.at[c, pl.ds(s * rows, rows)])


def _make_scat(n, e, d):
    per_tile = e // (_NC * _NS)
    tail = per_tile % _CHUNK
    n_pad = _rows_per_tile(n) * _NS
    scratch = [
        pltpu.VMEM_SHARED((n_pad, d), jnp.float32),
        pltpu.VMEM((_CHUNK,), jnp.int32),
        pltpu.VMEM((_CHUNK,), jnp.int32),
        pltpu.VMEM((max(tail, 8),), jnp.int32),
        pltpu.VMEM((max(tail, 8),), jnp.int32),
        pltpu.VMEM((_CHUNK, d), jnp.float32),
        pltpu.VMEM((max(tail, 8), d), jnp.float32),
        pltpu.SemaphoreType.DMA,
    ]
    return pl.kernel(
        functools.partial(_scat_body, n, e, d),
        out_type=jax.ShapeDtypeStruct((_NC, n_pad, d), jnp.float32),
        mesh=_MESH,
        scratch_types=scratch,
        compiler_params=pltpu.CompilerParams(use_tc_tiling_on_sc=False),
    )


def _prep_body(x_ref, w1_ref, b1_ref, w2_ref, b2_ref, d0_ref, d1_ref,
               u0_ref, ch_ref, c2_ref, sq_ref):
    x = x_ref[...]
    h1 = jax.nn.relu(
        lax.dot_general(x, w1_ref[...], (((1,), (1,)), ((), ())),
                        preferred_element_type=jnp.float32) + b1_ref[...])
    h2 = lax.dot_general(h1, w2_ref[...], (((1,), (1,)), ((), ())),
                         preferred_element_type=jnp.float32) + b2_ref[...]
    deg = d0_ref[...] + d1_ref[...] + 1.0
    dinv = lax.rsqrt(deg)
    u0 = h2 * dinv
    u0_ref[...] = u0
    ch_ref[...] = _ALPHA * u0
    c2_ref[...] = jnp.broadcast_to((1.0 - _ALPHA) * dinv * dinv, h2.shape)
    sq_ref[...] = jnp.broadcast_to(jnp.sqrt(deg), h2.shape)


def _tc_prep(x, w1, b1, w2, b2, d0, d1):
    n, fin = x.shape
    d = w2.shape[0]
    blk = 2000
    grid = (n // blk,)
    f32 = jnp.float32
    out_shape = [jax.ShapeDtypeStruct((n, d), f32)] * 4
    return pl.pallas_call(
        _prep_body,
        grid=grid,
        in_specs=[
            pl.BlockSpec((blk, fin), lambda i: (i, 0)),
            pl.BlockSpec(w1.shape, lambda i: (0, 0)),
            pl.BlockSpec((1, w1.shape[0]), lambda i: (0, 0)),
            pl.BlockSpec(w2.shape, lambda i: (0, 0)),
            pl.BlockSpec((1, w2.shape[0]), lambda i: (0, 0)),
            pl.BlockSpec((blk, 1), lambda i: (i, 0)),
            pl.BlockSpec((blk, 1), lambda i: (i, 0)),
        ],
        out_specs=[pl.BlockSpec((blk, d), lambda i: (i, 0))] * 4,
        out_shape=out_shape,
    )(x, w1, b1.reshape(1, -1), w2, b2.reshape(1, -1),
      d0.reshape(-1, 1), d1.reshape(-1, 1))


def _comb_body(s0_ref, s1_ref, u_ref, c2_ref, ch_ref, o_ref):
    o_ref[...] = (c2_ref[...] * (s0_ref[0] + s1_ref[0] + u_ref[...])
                  + ch_ref[...])


def _tc_combine(scatp, u, c2, ch):
    n, d = u.shape
    blk = 2000
    spec = pl.BlockSpec((blk, d), lambda i: (i, 0))
    spec3 = lambda c: pl.BlockSpec((1, blk, d), lambda i, _c=c: (_c, i, 0))
    return pl.pallas_call(
        _comb_body,
        grid=(n // blk,),
        in_specs=[spec3(0), spec3(1), spec, spec, spec],
        out_specs=spec,
        out_shape=jax.ShapeDtypeStruct((n, d), jnp.float32),
    )(scatp, scatp, u, c2, ch)


def _final_body(u_ref, sq_ref, o_ref):
    z = u_ref[...] * sq_ref[...]
    m = jnp.max(z, axis=1, keepdims=True)
    lse = jnp.log(jnp.sum(jnp.exp(z - m), axis=1, keepdims=True)) + m
    o_ref[...] = z - lse


def _tc_final(u, sq):
    n, d = u.shape
    blk = 2000
    spec = pl.BlockSpec((blk, d), lambda i: (i, 0))
    return pl.pallas_call(
        _final_body,
        grid=(n // blk,),
        in_specs=[spec] * 2,
        out_specs=spec,
        out_shape=jax.ShapeDtypeStruct((n, d), jnp.float32),
    )(u, sq)


def kernel(x, edge_index, W1, b1, W2, b2):
    n = x.shape[0]
    d = W2.shape[0]
    src = edge_index[0]
    dst = edge_index[1]

    degp = _sc_degree(dst, n)
    u0, ch, c2, sq = _tc_prep(x, W1, b1, W2, b2,
                              degp[0, :n, 0], degp[1, :n, 0])

    n_pad = _rows_per_tile(n) * _NS
    scat_fn = _make_scat(n, src.shape[0], d)
    zeros = jnp.zeros((n_pad, d), jnp.float32)
    u = u0
    for _ in range(_K):
        scatp = scat_fn(u, src, dst, zeros)
        u = _tc_combine(scatp, u, c2, ch)
    return _tc_final(u, sq)


# speedup vs baseline: 21.8458x; 1.7664x over previous
import functools

import jax
import jax.numpy as jnp
from jax import lax
from jax.experimental import pallas as pl
from jax.experimental.pallas import tpu as pltpu
from jax.experimental.pallas import tpu_sc as plsc

_ALPHA = 0.1
_K = 10
_CHUNK = 128

_NC = 2
_NS = 16
_MESH = plsc.VectorSubcoreMesh(core_axis_name="c", subcore_axis_name="s",
                               num_cores=_NC, num_subcores=_NS)


def _rows_per_tile(n):
    return -(-n // (_NS * 8)) * 8


def _deg_body(n, e, dst_hbm, zeros_hbm, ones_hbm, out_hbm,
              acc, ones_v, ones_t, idx_b, idx_t):
    c = lax.axis_index("c")
    s = lax.axis_index("s")
    rows = _rows_per_tile(n)
    per_tile = e // (_NC * _NS)
    n_chunks = per_tile // _CHUNK
    tail = per_tile - n_chunks * _CHUNK
    base = (c * _NS + s) * per_tile

    pltpu.sync_copy(zeros_hbm.at[pl.ds(s * rows, rows)],
                    acc.at[pl.ds(s * rows, rows)])
    pltpu.sync_copy(ones_hbm, ones_v)
    if tail:
        pltpu.sync_copy(ones_hbm.at[pl.ds(0, tail)], ones_t)
    plsc.subcore_barrier()

    def body(i, carry):
        off = pl.multiple_of(base + i * _CHUNK, 8)
        pltpu.sync_copy(dst_hbm.at[pl.ds(off, _CHUNK)], idx_b)
        pltpu.sync_copy(ones_v, acc.at[idx_b], add=True)
        return carry

    lax.fori_loop(0, n_chunks, body, 0)
    if tail:
        off = pl.multiple_of(base + n_chunks * _CHUNK, 8)
        pltpu.sync_copy(dst_hbm.at[pl.ds(off, tail)], idx_t)
        pltpu.sync_copy(ones_t, acc.at[idx_t], add=True)
    plsc.subcore_barrier()
    pltpu.sync_copy(acc.at[pl.ds(s * rows, rows)],
                    out_hbm.at[c, pl.ds(s * rows, rows)])


def _sc_degree(dst, n):
    e = dst.shape[0]
    per_tile = e // (_NC * _NS)
    tail = per_tile % _CHUNK
    n_pad = _rows_per_tile(n) * _NS
    zeros = jnp.zeros((n_pad, 8), jnp.float32)
    ones = jnp.ones((_CHUNK, 8), jnp.float32)
    scratch = [
        pltpu.VMEM_SHARED((n_pad, 8), jnp.float32),
        pltpu.VMEM((_CHUNK, 8), jnp.float32),
        pltpu.VMEM((max(tail, 8), 8), jnp.float32),
        pltpu.VMEM((_CHUNK,), jnp.int32),
        pltpu.VMEM((max(tail, 8),), jnp.int32),
    ]
    kfn = pl.kernel(
        functools.partial(_deg_body, n, e),
        out_type=jax.ShapeDtypeStruct((_NC, n_pad, 8), jnp.float32),
        mesh=_MESH,
        scratch_types=scratch,
        compiler_params=pltpu.CompilerParams(use_tc_tiling_on_sc=False),
    )
    return kfn(dst, zeros, ones)


def _scat_body(n, nch, d, ch, u_hbm, srcm_hbm, dstm_hbm, zeros_hbm, out_hbm,
               acc, src_v, dst_v, rbuf, gsem):
    c = lax.axis_index("c")
    s = lax.axis_index("s")
    rows = _rows_per_tile(n)
    row0 = (c * _NS + s) * nch

    pltpu.sync_copy(zeros_hbm.at[pl.ds(s * rows, rows)],
                    acc.at[pl.ds(s * rows, rows)])
    pltpu.sync_copy(srcm_hbm.at[pl.ds(row0, nch)], src_v)
    pltpu.sync_copy(dstm_hbm.at[pl.ds(row0, nch)], dst_v)
    pltpu.async_copy(u_hbm.at[src_v.at[0]], rbuf.at[0], gsem)
    plsc.subcore_barrier()

    def body(j, carry):
        cur = lax.rem(j, 2)
        pltpu.make_async_copy(u_hbm.at[src_v.at[j]], rbuf.at[cur], gsem).wait()

        @pl.when(j + 1 < nch)
        def _():
            pltpu.async_copy(u_hbm.at[src_v.at[j + 1]],
                             rbuf.at[lax.rem(j + 1, 2)], gsem)

        pltpu.sync_copy(rbuf.at[cur], acc.at[dst_v.at[j]], add=True)
        return carry

    lax.fori_loop(0, nch, body, 0)
    plsc.subcore_barrier()
    pltpu.sync_copy(acc.at[pl.ds(s * rows, rows)],
                    out_hbm.at[c, pl.ds(s * rows, rows)])


def _make_scat(n, e, d, ch):
    assert e % (_NC * _NS * ch) == 0
    nch = e // (_NC * _NS * ch)
    n_pad = _rows_per_tile(n) * _NS
    scratch = [
        pltpu.VMEM_SHARED((n_pad, d), jnp.float32),
        pltpu.VMEM((nch, ch), jnp.int32),
        pltpu.VMEM((nch, ch), jnp.int32),
        pltpu.VMEM((2, ch, d), jnp.float32),
        pltpu.SemaphoreType.DMA,
    ]
    return pl.kernel(
        functools.partial(_scat_body, n, nch, d, ch),
        out_type=jax.ShapeDtypeStruct((_NC, n_pad, d), jnp.float32),
        mesh=_MESH,
        scratch_types=scratch,
        compiler_params=pltpu.CompilerParams(use_tc_tiling_on_sc=False),
    )


def _prep_body(x_ref, w1_ref, b1_ref, w2_ref, b2_ref, d0_ref, d1_ref,
               u0_ref, ch_ref, c2_ref, sq_ref):
    x = x_ref[...]
    h1 = jax.nn.relu(
        lax.dot_general(x, w1_ref[...], (((1,), (1,)), ((), ())),
                        preferred_element_type=jnp.float32) + b1_ref[...])
    h2 = lax.dot_general(h1, w2_ref[...], (((1,), (1,)), ((), ())),
                         preferred_element_type=jnp.float32) + b2_ref[...]
    deg = d0_ref[...] + d1_ref[...] + 1.0
    dinv = lax.rsqrt(deg)
    u0 = h2 * dinv
    u0_ref[...] = u0
    ch_ref[...] = _ALPHA * u0
    c2_ref[...] = jnp.broadcast_to((1.0 - _ALPHA) * dinv * dinv, h2.shape)
    sq_ref[...] = jnp.broadcast_to(jnp.sqrt(deg), h2.shape)


def _tc_prep(x, w1, b1, w2, b2, d0, d1):
    n, fin = x.shape
    d = w2.shape[0]
    blk = 2000
    grid = (n // blk,)
    f32 = jnp.float32
    out_shape = [jax.ShapeDtypeStruct((n, d), f32)] * 4
    return pl.pallas_call(
        _prep_body,
        grid=grid,
        in_specs=[
            pl.BlockSpec((blk, fin), lambda i: (i, 0)),
            pl.BlockSpec(w1.shape, lambda i: (0, 0)),
            pl.BlockSpec((1, w1.shape[0]), lambda i: (0, 0)),
            pl.BlockSpec(w2.shape, lambda i: (0, 0)),
            pl.BlockSpec((1, w2.shape[0]), lambda i: (0, 0)),
            pl.BlockSpec((blk, 1), lambda i: (i, 0)),
            pl.BlockSpec((blk, 1), lambda i: (i, 0)),
        ],
        out_specs=[pl.BlockSpec((blk, d), lambda i: (i, 0))] * 4,
        out_shape=out_shape,
    )(x, w1, b1.reshape(1, -1), w2, b2.reshape(1, -1),
      d0.reshape(-1, 1), d1.reshape(-1, 1))


def _comb_body(s0_ref, s1_ref, u_ref, c2_ref, ch_ref, o_ref):
    o_ref[...] = (c2_ref[...] * (s0_ref[0] + s1_ref[0] + u_ref[...])
                  + ch_ref[...])


def _tc_combine(scatp, u, c2, ch):
    n, d = u.shape
    blk = 2000
    spec = pl.BlockSpec((blk, d), lambda i: (i, 0))
    spec3 = lambda c: pl.BlockSpec((1, blk, d), lambda i, _c=c: (_c, i, 0))
    return pl.pallas_call(
        _comb_body,
        grid=(n // blk,),
        in_specs=[spec3(0), spec3(1), spec, spec, spec],
        out_specs=spec,
        out_shape=jax.ShapeDtypeStruct((n, d), jnp.float32),
    )(scatp, scatp, u, c2, ch)


def _final_body(u_ref, sq_ref, o_ref):
    z = u_ref[...] * sq_ref[...]
    m = jnp.max(z, axis=1, keepdims=True)
    lse = jnp.log(jnp.sum(jnp.exp(z - m), axis=1, keepdims=True)) + m
    o_ref[...] = z - lse


def _tc_final(u, sq):
    n, d = u.shape
    blk = 2000
    spec = pl.BlockSpec((blk, d), lambda i: (i, 0))
    return pl.pallas_call(
        _final_body,
        grid=(n // blk,),
        in_specs=[spec] * 2,
        out_specs=spec,
        out_shape=jax.ShapeDtypeStruct((n, d), jnp.float32),
    )(u, sq)


def kernel(x, edge_index, W1, b1, W2, b2):
    n = x.shape[0]
    d = W2.shape[0]
    src = edge_index[0]
    dst = edge_index[1]

    degp = _sc_degree(dst, n)
    u0, ch, c2, sq = _tc_prep(x, W1, b1, W2, b2,
                              degp[0, :n, 0], degp[1, :n, 0])

    n_pad = _rows_per_tile(n) * _NS
    e = src.shape[0]
    chunk = 125
    scat_fn = _make_scat(n, e, d, chunk)
    srcm = src.reshape(-1, chunk)
    dstm = dst.reshape(-1, chunk)
    zeros = jnp.zeros((n_pad, d), jnp.float32)
    u = u0
    for _ in range(_K):
        scatp = scat_fn(u, srcm, dstm, zeros)
        u = _tc_combine(scatp, u, c2, ch)
    return _tc_final(u, sq)


# speedup vs baseline: 30.2519x; 1.3848x over previous
import functools

import jax
import jax.numpy as jnp
from jax import lax
from jax.experimental import pallas as pl
from jax.experimental.pallas import tpu as pltpu
from jax.experimental.pallas import tpu_sc as plsc

_ALPHA = 0.1
_K = 10
_CHUNK = 128

_NC = 2
_NS = 16
_MESH = plsc.VectorSubcoreMesh(core_axis_name="c", subcore_axis_name="s",
                               num_cores=_NC, num_subcores=_NS)


def _rows_per_tile(n):
    return -(-n // (_NS * 8)) * 8


def _deg_body(n, nch, ch, dstm_hbm, zeros_hbm, ones_hbm, out_hbm,
              acc, ones_v, dst_v, ssem):
    c = lax.axis_index("c")
    s = lax.axis_index("s")
    rows = _rows_per_tile(n)
    row0 = (c * _NS + s) * nch

    pltpu.sync_copy(zeros_hbm.at[pl.ds(s * rows, rows)],
                    acc.at[pl.ds(s * rows, rows)])
    pltpu.sync_copy(ones_hbm, ones_v)
    pltpu.sync_copy(dstm_hbm.at[pl.ds(row0, nch)], dst_v)
    plsc.subcore_barrier()

    def body(j, carry):
        @pl.when(j >= 2)
        def _():
            pltpu.make_async_copy(ones_v, acc.at[dst_v.at[j - 2]],
                                  ssem).wait()

        pltpu.async_copy(ones_v, acc.at[dst_v.at[j]], ssem, add=True)
        return carry

    lax.fori_loop(0, nch, body, 0)
    for t in range(max(nch - 2, 0), nch):
        pltpu.make_async_copy(ones_v, acc.at[dst_v.at[t]], ssem).wait()
    plsc.subcore_barrier()
    pltpu.sync_copy(acc.at[pl.ds(s * rows, rows)],
                    out_hbm.at[c, pl.ds(s * rows, rows)])


def _sc_degree(dstm, n):
    nrows, ch = dstm.shape
    nch = nrows // (_NC * _NS)
    n_pad = _rows_per_tile(n) * _NS
    zeros = jnp.zeros((n_pad, 8), jnp.float32)
    ones = jnp.ones((ch, 8), jnp.float32)
    scratch = [
        pltpu.VMEM_SHARED((n_pad, 8), jnp.float32),
        pltpu.VMEM((ch, 8), jnp.float32),
        pltpu.VMEM((nch, ch), jnp.int32),
        pltpu.SemaphoreType.DMA,
    ]
    kfn = pl.kernel(
        functools.partial(_deg_body, n, nch, ch),
        out_type=jax.ShapeDtypeStruct((_NC, n_pad, 8), jnp.float32),
        mesh=_MESH,
        scratch_types=scratch,
        compiler_params=pltpu.CompilerParams(use_tc_tiling_on_sc=False),
    )
    return kfn(dstm, zeros, ones)


def _scat_body(n, nch, d, ch, u_hbm, srcm_hbm, dstm_hbm, zeros_hbm, out_hbm,
               acc, src_v, dst_v, rbuf, gsem, ssem):
    c = lax.axis_index("c")
    s = lax.axis_index("s")
    rows = _rows_per_tile(n)
    row0 = (c * _NS + s) * nch

    pltpu.sync_copy(zeros_hbm.at[pl.ds(s * rows, rows)],
                    acc.at[pl.ds(s * rows, rows)])
    pltpu.sync_copy(srcm_hbm.at[pl.ds(row0, nch)], src_v)
    pltpu.sync_copy(dstm_hbm.at[pl.ds(row0, nch)], dst_v)
    pltpu.async_copy(u_hbm.at[src_v.at[0]], rbuf.at[0], gsem)
    plsc.subcore_barrier()

    def body(j, carry):
        @pl.when(j >= 3)
        def _():
            pltpu.make_async_copy(rbuf.at[lax.rem(j - 3, 4)],
                                  acc.at[dst_v.at[j - 3]], ssem).wait()

        @pl.when(j + 1 < nch)
        def _():
            pltpu.async_copy(u_hbm.at[src_v.at[j + 1]],
                             rbuf.at[lax.rem(j + 1, 4)], gsem)

        cur = lax.rem(j, 4)
        pltpu.make_async_copy(u_hbm.at[src_v.at[j]], rbuf.at[cur],
                              gsem).wait()
        pltpu.async_copy(rbuf.at[cur], acc.at[dst_v.at[j]], ssem, add=True)
        return carry

    lax.fori_loop(0, nch, body, 0)
    for t in range(max(nch - 3, 0), nch):
        pltpu.make_async_copy(rbuf.at[t % 4], acc.at[dst_v.at[t]],
                              ssem).wait()
    plsc.subcore_barrier()
    pltpu.sync_copy(acc.at[pl.ds(s * rows, rows)],
                    out_hbm.at[c, pl.ds(s * rows, rows)])


def _make_scat(n, e, d, ch):
    assert e % (_NC * _NS * ch) == 0
    nch = e // (_NC * _NS * ch)
    n_pad = _rows_per_tile(n) * _NS
    scratch = [
        pltpu.VMEM_SHARED((n_pad, d), jnp.float32),
        pltpu.VMEM((nch, ch), jnp.int32),
        pltpu.VMEM((nch, ch), jnp.int32),
        pltpu.VMEM((4, ch, d), jnp.float32),
        pltpu.SemaphoreType.DMA,
        pltpu.SemaphoreType.DMA,
    ]
    return pl.kernel(
        functools.partial(_scat_body, n, nch, d, ch),
        out_type=jax.ShapeDtypeStruct((_NC, n_pad, d), jnp.float32),
        mesh=_MESH,
        scratch_types=scratch,
        compiler_params=pltpu.CompilerParams(use_tc_tiling_on_sc=False),
    )


def _prep_body(x_ref, w1_ref, b1_ref, w2_ref, b2_ref, d0_ref, d1_ref,
               u0_ref, ch_ref, c2_ref, sq_ref):
    x = x_ref[...]
    h1 = jax.nn.relu(
        lax.dot_general(x, w1_ref[...], (((1,), (1,)), ((), ())),
                        preferred_element_type=jnp.float32) + b1_ref[...])
    h2 = lax.dot_general(h1, w2_ref[...], (((1,), (1,)), ((), ())),
                         preferred_element_type=jnp.float32) + b2_ref[...]
    deg = d0_ref[...] + d1_ref[...] + 1.0
    dinv = lax.rsqrt(deg)
    u0 = h2 * dinv
    u0_ref[...] = u0
    ch_ref[...] = _ALPHA * u0
    c2_ref[...] = jnp.broadcast_to((1.0 - _ALPHA) * dinv * dinv, h2.shape)
    sq_ref[...] = jnp.broadcast_to(jnp.sqrt(deg), h2.shape)


def _tc_prep(x, w1, b1, w2, b2, d0, d1):
    n, fin = x.shape
    d = w2.shape[0]
    blk = 2000
    grid = (n // blk,)
    f32 = jnp.float32
    out_shape = [jax.ShapeDtypeStruct((n, d), f32)] * 4
    return pl.pallas_call(
        _prep_body,
        grid=grid,
        in_specs=[
            pl.BlockSpec((blk, fin), lambda i: (i, 0)),
            pl.BlockSpec(w1.shape, lambda i: (0, 0)),
            pl.BlockSpec((1, w1.shape[0]), lambda i: (0, 0)),
            pl.BlockSpec(w2.shape, lambda i: (0, 0)),
            pl.BlockSpec((1, w2.shape[0]), lambda i: (0, 0)),
            pl.BlockSpec((blk, 1), lambda i: (i, 0)),
            pl.BlockSpec((blk, 1), lambda i: (i, 0)),
        ],
        out_specs=[pl.BlockSpec((blk, d), lambda i: (i, 0))] * 4,
        out_shape=out_shape,
    )(x, w1, b1.reshape(1, -1), w2, b2.reshape(1, -1),
      d0.reshape(-1, 1), d1.reshape(-1, 1))


def _comb_body(s0_ref, s1_ref, u_ref, c2_ref, ch_ref, o_ref):
    o_ref[...] = (c2_ref[...] * (s0_ref[0] + s1_ref[0] + u_ref[...])
                  + ch_ref[...])


def _tc_combine(scatp, u, c2, ch):
    n, d = u.shape
    blk = 2000
    spec = pl.BlockSpec((blk, d), lambda i: (i, 0))
    spec3 = lambda c: pl.BlockSpec((1, blk, d), lambda i, _c=c: (_c, i, 0))
    return pl.pallas_call(
        _comb_body,
        grid=(n // blk,),
        in_specs=[spec3(0), spec3(1), spec, spec, spec],
        out_specs=spec,
        out_shape=jax.ShapeDtypeStruct((n, d), jnp.float32),
    )(scatp, scatp, u, c2, ch)


def _final_body(u_ref, sq_ref, o_ref):
    z = u_ref[...] * sq_ref[...]
    m = jnp.max(z, axis=1, keepdims=True)
    lse = jnp.log(jnp.sum(jnp.exp(z - m), axis=1, keepdims=True)) + m
    o_ref[...] = z - lse


def _tc_final(u, sq):
    n, d = u.shape
    blk = 2000
    spec = pl.BlockSpec((blk, d), lambda i: (i, 0))
    return pl.pallas_call(
        _final_body,
        grid=(n // blk,),
        in_specs=[spec] * 2,
        out_specs=spec,
        out_shape=jax.ShapeDtypeStruct((n, d), jnp.float32),
    )(u, sq)


def kernel(x, edge_index, W1, b1, W2, b2):
    n = x.shape[0]
    d = W2.shape[0]
    src = edge_index[0]
    dst = edge_index[1]

    e = src.shape[0]
    chunk = 125
    srcm = src.reshape(-1, chunk)
    dstm = dst.reshape(-1, chunk)

    degp = _sc_degree(dstm, n)
    u0, ch, c2, sq = _tc_prep(x, W1, b1, W2, b2,
                              degp[0, :n, 0], degp[1, :n, 0])

    n_pad = _rows_per_tile(n) * _NS
    scat_fn = _make_scat(n, e, d, chunk)
    zeros = jnp.zeros((n_pad, d), jnp.float32)
    u = u0
    for _ in range(_K):
        scatp = scat_fn(u, srcm, dstm, zeros)
        u = _tc_combine(scatp, u, c2, ch)
    return _tc_final(u, sq)


# speedup vs baseline: 36.9368x; 1.2210x over previous
import functools

import jax
import jax.numpy as jnp
from jax import lax
from jax.experimental import pallas as pl
from jax.experimental.pallas import tpu as pltpu
from jax.experimental.pallas import tpu_sc as plsc

_ALPHA = 0.1
_K = 10
_CHUNK = 128

_NC = 2
_NS = 16
_MESH = plsc.VectorSubcoreMesh(core_axis_name="c", subcore_axis_name="s",
                               num_cores=_NC, num_subcores=_NS)


def _rows_per_tile(n):
    return -(-n // (_NS * 8)) * 8


def _deg_body(n, nch, ch, dstm_hbm, zeros_hbm, ones_hbm, out_hbm,
              acc, ones_v, dst_v, ssem):
    c = lax.axis_index("c")
    s = lax.axis_index("s")
    rows = _rows_per_tile(n)
    row0 = (c * _NS + s) * nch

    pltpu.sync_copy(zeros_hbm.at[pl.ds(s * rows, rows)],
                    acc.at[pl.ds(s * rows, rows)])
    pltpu.sync_copy(ones_hbm, ones_v)
    pltpu.sync_copy(dstm_hbm.at[pl.ds(row0, nch)], dst_v)
    plsc.subcore_barrier()

    def body(j, carry):
        @pl.when(j >= 2)
        def _():
            pltpu.make_async_copy(ones_v, acc.at[dst_v.at[j - 2]],
                                  ssem).wait()

        pltpu.async_copy(ones_v, acc.at[dst_v.at[j]], ssem, add=True)
        return carry

    lax.fori_loop(0, nch, body, 0)
    for t in range(max(nch - 2, 0), nch):
        pltpu.make_async_copy(ones_v, acc.at[dst_v.at[t]], ssem).wait()
    plsc.subcore_barrier()
    pltpu.sync_copy(acc.at[pl.ds(s * rows, rows)],
                    out_hbm.at[c, pl.ds(s * rows, rows)])


def _sc_degree(dstm, n):
    nrows, ch = dstm.shape
    nch = nrows // (_NC * _NS)
    n_pad = _rows_per_tile(n) * _NS
    zeros = jnp.zeros((n_pad, 8), jnp.float32)
    ones = jnp.ones((ch, 8), jnp.float32)
    scratch = [
        pltpu.VMEM_SHARED((n_pad, 8), jnp.float32),
        pltpu.VMEM((ch, 8), jnp.float32),
        pltpu.VMEM((nch, ch), jnp.int32),
        pltpu.SemaphoreType.DMA,
    ]
    kfn = pl.kernel(
        functools.partial(_deg_body, n, nch, ch),
        out_type=jax.ShapeDtypeStruct((_NC, n_pad, 8), jnp.float32),
        mesh=_MESH,
        scratch_types=scratch,
        compiler_params=pltpu.CompilerParams(use_tc_tiling_on_sc=False),
    )
    return kfn(dstm, zeros, ones)


def _scat_body(n, nch, d, ch, u_hbm, srcm_hbm, dstm_hbm, zeros_hbm, out_hbm,
               acc, src_v, dst_v, rbuf, gsem, ssem):
    c = lax.axis_index("c")
    s = lax.axis_index("s")
    rows = _rows_per_tile(n)
    row0 = (c * _NS + s) * nch

    pltpu.sync_copy(zeros_hbm.at[pl.ds(s * rows, rows)],
                    acc.at[pl.ds(s * rows, rows)])
    pltpu.sync_copy(srcm_hbm.at[pl.ds(row0, nch)], src_v)
    pltpu.sync_copy(dstm_hbm.at[pl.ds(row0, nch)], dst_v)
    pltpu.async_copy(u_hbm.at[src_v.at[0]], rbuf.at[0], gsem)
    plsc.subcore_barrier()

    if nch > 2:
        pltpu.async_copy(u_hbm.at[src_v.at[1]], rbuf.at[1], gsem)
        pltpu.async_copy(u_hbm.at[src_v.at[2]], rbuf.at[2], gsem)

    look = 3 if nch > 2 else 1

    def body(j, carry):
        @pl.when(j >= 8 - look)
        def _():
            pltpu.make_async_copy(rbuf.at[lax.rem(j - (8 - look), 8)],
                                  acc.at[dst_v.at[j - (8 - look)]],
                                  ssem).wait()

        @pl.when(j + look < nch)
        def _():
            pltpu.async_copy(u_hbm.at[src_v.at[j + look]],
                             rbuf.at[lax.rem(j + look, 8)], gsem)

        cur = lax.rem(j, 8)
        pltpu.make_async_copy(u_hbm.at[src_v.at[j]], rbuf.at[cur],
                              gsem).wait()
        pltpu.async_copy(rbuf.at[cur], acc.at[dst_v.at[j]], ssem, add=True)
        return carry

    lax.fori_loop(0, nch, body, 0)
    for t in range(max(nch - (8 - look), 0), nch):
        pltpu.make_async_copy(rbuf.at[t % 8], acc.at[dst_v.at[t]],
                              ssem).wait()
    plsc.subcore_barrier()
    pltpu.sync_copy(acc.at[pl.ds(s * rows, rows)],
                    out_hbm.at[c, pl.ds(s * rows, rows)])


def _make_scat(n, e, d, ch):
    assert e % (_NC * _NS * ch) == 0
    nch = e // (_NC * _NS * ch)
    n_pad = _rows_per_tile(n) * _NS
    scratch = [
        pltpu.VMEM_SHARED((n_pad, d), jnp.float32),
        pltpu.VMEM((nch, ch), jnp.int32),
        pltpu.VMEM((nch, ch), jnp.int32),
        pltpu.VMEM((8, ch, d), jnp.float32),
        pltpu.SemaphoreType.DMA,
        pltpu.SemaphoreType.DMA,
    ]
    return pl.kernel(
        functools.partial(_scat_body, n, nch, d, ch),
        out_type=jax.ShapeDtypeStruct((_NC, n_pad, d), jnp.float32),
        mesh=_MESH,
        scratch_types=scratch,
        compiler_params=pltpu.CompilerParams(use_tc_tiling_on_sc=False),
    )


def _prep_body(x_ref, w1_ref, b1_ref, w2_ref, b2_ref, d0_ref, d1_ref,
               u0_ref, ch_ref, c2_ref, sq_ref):
    x = x_ref[...]
    h1 = jax.nn.relu(
        lax.dot_general(x, w1_ref[...], (((1,), (1,)), ((), ())),
                        preferred_element_type=jnp.float32) + b1_ref[...])
    h2 = lax.dot_general(h1, w2_ref[...], (((1,), (1,)), ((), ())),
                         preferred_element_type=jnp.float32) + b2_ref[...]
    deg = d0_ref[...] + d1_ref[...] + 1.0
    dinv = lax.rsqrt(deg)
    u0 = h2 * dinv
    u0_ref[...] = u0
    ch_ref[...] = _ALPHA * u0
    c2_ref[...] = jnp.broadcast_to((1.0 - _ALPHA) * dinv * dinv, h2.shape)
    sq_ref[...] = jnp.broadcast_to(jnp.sqrt(deg), h2.shape)


def _tc_prep(x, w1, b1, w2, b2, d0, d1):
    n, fin = x.shape
    d = w2.shape[0]
    blk = 2000
    grid = (n // blk,)
    f32 = jnp.float32
    out_shape = [jax.ShapeDtypeStruct((n, d), f32)] * 4
    return pl.pallas_call(
        _prep_body,
        grid=grid,
        in_specs=[
            pl.BlockSpec((blk, fin), lambda i: (i, 0)),
            pl.BlockSpec(w1.shape, lambda i: (0, 0)),
            pl.BlockSpec((1, w1.shape[0]), lambda i: (0, 0)),
            pl.BlockSpec(w2.shape, lambda i: (0, 0)),
            pl.BlockSpec((1, w2.shape[0]), lambda i: (0, 0)),
            pl.BlockSpec((blk, 1), lambda i: (i, 0)),
            pl.BlockSpec((blk, 1), lambda i: (i, 0)),
        ],
        out_specs=[pl.BlockSpec((blk, d), lambda i: (i, 0))] * 4,
        out_shape=out_shape,
    )(x, w1, b1.reshape(1, -1), w2, b2.reshape(1, -1),
      d0.reshape(-1, 1), d1.reshape(-1, 1))


def _comb_body(n_pad, d, scatp_hbm, u_hbm, c2_hbm, ch_hbm, uo_hbm,
               s0v, s1v, uv, c2v, chv, ov, sem):
    c = lax.axis_index("c")
    s = lax.axis_index("s")
    stripe = n_pad // (_NC * _NS)
    r0 = (c * _NS + s) * stripe
    pltpu.async_copy(scatp_hbm.at[0, pl.ds(r0, stripe)], s0v, sem)
    pltpu.async_copy(scatp_hbm.at[1, pl.ds(r0, stripe)], s1v, sem)
    pltpu.async_copy(u_hbm.at[pl.ds(r0, stripe)], uv, sem)
    pltpu.async_copy(c2_hbm.at[pl.ds(r0, stripe)], c2v, sem)
    pltpu.async_copy(ch_hbm.at[pl.ds(r0, stripe)], chv, sem)
    pltpu.make_async_copy(scatp_hbm.at[0, pl.ds(r0, stripe)], s0v, sem).wait()
    pltpu.make_async_copy(scatp_hbm.at[1, pl.ds(r0, stripe)], s1v, sem).wait()
    pltpu.make_async_copy(u_hbm.at[pl.ds(r0, stripe)], uv, sem).wait()
    pltpu.make_async_copy(c2_hbm.at[pl.ds(r0, stripe)], c2v, sem).wait()
    pltpu.make_async_copy(ch_hbm.at[pl.ds(r0, stripe)], chv, sem).wait()

    def rowb(r, carry):
        for k in range(d // 16):
            sl = (r, pl.ds(k * 16, 16))
            ov[sl] = c2v[sl] * (s0v[sl] + s1v[sl] + uv[sl]) + chv[sl]
        return carry

    lax.fori_loop(0, stripe, rowb, 0)
    pltpu.sync_copy(ov, uo_hbm.at[pl.ds(r0, stripe)])


def _make_combine(n_pad, d):
    stripe = n_pad // (_NC * _NS)
    buf = lambda: pltpu.VMEM((stripe, d), jnp.float32)
    scratch = [buf(), buf(), buf(), buf(), buf(), buf(),
               pltpu.SemaphoreType.DMA]
    return pl.kernel(
        functools.partial(_comb_body, n_pad, d),
        out_type=jax.ShapeDtypeStruct((n_pad, d), jnp.float32),
        mesh=_MESH,
        scratch_types=scratch,
        compiler_params=pltpu.CompilerParams(use_tc_tiling_on_sc=False),
    )


def _final_body(u_ref, sq_ref, o_ref):
    z = u_ref[...] * sq_ref[...]
    m = jnp.max(z, axis=1, keepdims=True)
    lse = jnp.log(jnp.sum(jnp.exp(z - m), axis=1, keepdims=True)) + m
    o_ref[...] = z - lse


def _tc_final(u, sq):
    n, d = u.shape
    blk = 2000
    spec = pl.BlockSpec((blk, d), lambda i: (i, 0))
    return pl.pallas_call(
        _final_body,
        grid=(n // blk,),
        in_specs=[spec] * 2,
        out_specs=spec,
        out_shape=jax.ShapeDtypeStruct((n, d), jnp.float32),
    )(u, sq)


def kernel(x, edge_index, W1, b1, W2, b2):
    n = x.shape[0]
    d = W2.shape[0]
    src = edge_index[0]
    dst = edge_index[1]

    e = src.shape[0]
    chunk = 125
    srcm = src.reshape(-1, chunk)
    dstm = dst.reshape(-1, chunk)

    degp = _sc_degree(dstm, n)
    u0, ch, c2, sq = _tc_prep(x, W1, b1, W2, b2,
                              degp[0, :n, 0], degp[1, :n, 0])

    n_pad = _rows_per_tile(n) * _NS
    pad = ((0, n_pad - n), (0, 0))
    u = jnp.pad(u0, pad)
    chp = jnp.pad(ch, pad)
    c2p = jnp.pad(c2, pad)

    scat_fn = _make_scat(n, e, d, chunk)
    comb_fn = _make_combine(n_pad, d)
    zeros = jnp.zeros((n_pad, d), jnp.float32)
    for _ in range(_K):
        scatp = scat_fn(u, srcm, dstm, zeros)
        u = comb_fn(scatp, u, c2p, chp)
    return _tc_final(u[:n], sq)


# speedup vs baseline: 36.9434x; 1.0002x over previous
import functools

import jax
import jax.numpy as jnp
from jax import lax
from jax.experimental import pallas as pl
from jax.experimental.pallas import tpu as pltpu
from jax.experimental.pallas import tpu_sc as plsc

_ALPHA = 0.1
_K = 10
_CHUNK = 128

_NC = 2
_NS = 16
_MESH = plsc.VectorSubcoreMesh(core_axis_name="c", subcore_axis_name="s",
                               num_cores=_NC, num_subcores=_NS)


def _rows_per_tile(n):
    return -(-n // (_NS * 8)) * 8


def _deg_body(n, nch, ch, dstm_hbm, zeros_hbm, ones_hbm, out_hbm,
              acc, ones_v, dst_v, ssem):
    c = lax.axis_index("c")
    s = lax.axis_index("s")
    rows = _rows_per_tile(n)
    row0 = (c * _NS + s) * nch

    pltpu.sync_copy(zeros_hbm.at[pl.ds(s * rows, rows)],
                    acc.at[pl.ds(s * rows, rows)])
    pltpu.sync_copy(ones_hbm, ones_v)
    pltpu.sync_copy(dstm_hbm.at[pl.ds(row0, nch)], dst_v)
    plsc.subcore_barrier()

    def body(j, carry):
        @pl.when(j >= 2)
        def _():
            pltpu.make_async_copy(ones_v, acc.at[dst_v.at[j - 2]],
                                  ssem).wait()

        pltpu.async_copy(ones_v, acc.at[dst_v.at[j]], ssem, add=True)
        return carry

    lax.fori_loop(0, nch, body, 0)
    for t in range(max(nch - 2, 0), nch):
        pltpu.make_async_copy(ones_v, acc.at[dst_v.at[t]], ssem).wait()
    plsc.subcore_barrier()
    pltpu.sync_copy(acc.at[pl.ds(s * rows, rows)],
                    out_hbm.at[c, pl.ds(s * rows, rows)])


def _sc_degree(dstm, n):
    nrows, ch = dstm.shape
    nch = nrows // (_NC * _NS)
    n_pad = _rows_per_tile(n) * _NS
    zeros = jnp.zeros((n_pad, 8), jnp.float32)
    ones = jnp.ones((ch, 8), jnp.float32)
    scratch = [
        pltpu.VMEM_SHARED((n_pad, 8), jnp.float32),
        pltpu.VMEM((ch, 8), jnp.float32),
        pltpu.VMEM((nch, ch), jnp.int32),
        pltpu.SemaphoreType.DMA,
    ]
    kfn = pl.kernel(
        functools.partial(_deg_body, n, nch, ch),
        out_type=jax.ShapeDtypeStruct((_NC, n_pad, 8), jnp.float32),
        mesh=_MESH,
        scratch_types=scratch,
        compiler_params=pltpu.CompilerParams(use_tc_tiling_on_sc=False),
    )
    return kfn(dstm, zeros, ones)


def _scat_body(n, nch, d, ch, u_hbm, srcm_hbm, dstm_hbm, zeros_hbm, out_hbm,
               acc, src_v, dst_v, rbuf, gsem, ssem):
    c = lax.axis_index("c")
    s = lax.axis_index("s")
    rows = _rows_per_tile(n)
    row0 = (c * _NS + s) * nch

    pltpu.sync_copy(zeros_hbm.at[pl.ds(s * rows, rows)],
                    acc.at[pl.ds(s * rows, rows)])
    pltpu.sync_copy(srcm_hbm.at[pl.ds(row0, nch)], src_v)
    pltpu.sync_copy(dstm_hbm.at[pl.ds(row0, nch)], dst_v)
    pltpu.async_copy(u_hbm.at[src_v.at[0]], rbuf.at[0], gsem)
    plsc.subcore_barrier()

    if nch > 2:
        pltpu.async_copy(u_hbm.at[src_v.at[1]], rbuf.at[1], gsem)
        pltpu.async_copy(u_hbm.at[src_v.at[2]], rbuf.at[2], gsem)

    look = 3 if nch > 2 else 1

    def body(j, carry):
        @pl.when(j >= 8 - look)
        def _():
            pltpu.make_async_copy(rbuf.at[lax.rem(j - (8 - look), 8)],
                                  acc.at[dst_v.at[j - (8 - look)]],
                                  ssem).wait()

        @pl.when(j + look < nch)
        def _():
            pltpu.async_copy(u_hbm.at[src_v.at[j + look]],
                             rbuf.at[lax.rem(j + look, 8)], gsem)

        cur = lax.rem(j, 8)
        pltpu.make_async_copy(u_hbm.at[src_v.at[j]], rbuf.at[cur],
                              gsem).wait()
        pltpu.async_copy(rbuf.at[cur], acc.at[dst_v.at[j]], ssem, add=True)
        return carry

    lax.fori_loop(0, nch, body, 0)
    for t in range(max(nch - (8 - look), 0), nch):
        pltpu.make_async_copy(rbuf.at[t % 8], acc.at[dst_v.at[t]],
                              ssem).wait()
    plsc.subcore_barrier()
    pltpu.sync_copy(acc.at[pl.ds(s * rows, rows)],
                    out_hbm.at[c, pl.ds(s * rows, rows)])


def _make_scat(n, e, d, ch):
    assert e % (_NC * _NS * ch) == 0
    nch = e // (_NC * _NS * ch)
    n_pad = _rows_per_tile(n) * _NS
    scratch = [
        pltpu.VMEM_SHARED((n_pad, d), jnp.float32),
        pltpu.VMEM((nch, ch), jnp.int32),
        pltpu.VMEM((nch, ch), jnp.int32),
        pltpu.VMEM((8, ch, d), jnp.float32),
        pltpu.SemaphoreType.DMA,
        pltpu.SemaphoreType.DMA,
    ]
    return pl.kernel(
        functools.partial(_scat_body, n, nch, d, ch),
        out_type=jax.ShapeDtypeStruct((_NC, n_pad, d), jnp.float32),
        mesh=_MESH,
        scratch_types=scratch,
        compiler_params=pltpu.CompilerParams(use_tc_tiling_on_sc=False),
    )


def _prep_body(x_ref, w1_ref, b1_ref, w2_ref, b2_ref, d0_ref, d1_ref,
               u0_ref, ch_ref, c2_ref, sq_ref):
    x = x_ref[...]
    h1 = jax.nn.relu(
        lax.dot_general(x, w1_ref[...], (((1,), (1,)), ((), ())),
                        preferred_element_type=jnp.float32) + b1_ref[...])
    h2 = lax.dot_general(h1, w2_ref[...], (((1,), (1,)), ((), ())),
                         preferred_element_type=jnp.float32) + b2_ref[...]
    deg = d0_ref[...] + d1_ref[...] + 1.0
    dinv = lax.rsqrt(deg)
    u0 = h2 * dinv
    u0_ref[...] = u0
    ch_ref[...] = _ALPHA * u0
    c2_ref[...] = jnp.broadcast_to((1.0 - _ALPHA) * dinv * dinv, h2.shape)
    sq_ref[...] = jnp.broadcast_to(jnp.sqrt(deg), h2.shape)


def _tc_prep(x, w1, b1, w2, b2, d0, d1, n_pad):
    n, fin = x.shape
    d = w2.shape[0]
    blk = 2000
    grid = (n // blk,)
    f32 = jnp.float32
    out_shape = [jax.ShapeDtypeStruct((n_pad, d), f32)] * 3 + [
        jax.ShapeDtypeStruct((n, d), f32)]
    return pl.pallas_call(
        _prep_body,
        grid=grid,
        in_specs=[
            pl.BlockSpec((blk, fin), lambda i: (i, 0)),
            pl.BlockSpec(w1.shape, lambda i: (0, 0)),
            pl.BlockSpec((1, w1.shape[0]), lambda i: (0, 0)),
            pl.BlockSpec(w2.shape, lambda i: (0, 0)),
            pl.BlockSpec((1, w2.shape[0]), lambda i: (0, 0)),
            pl.BlockSpec((blk, 1), lambda i: (i, 0)),
            pl.BlockSpec((blk, 1), lambda i: (i, 0)),
        ],
        out_specs=[pl.BlockSpec((blk, d), lambda i: (i, 0))] * 4,
        out_shape=out_shape,
    )(x, w1, b1.reshape(1, -1), w2, b2.reshape(1, -1),
      d0.reshape(-1, 1), d1.reshape(-1, 1))


def _comb_body(n_pad, d, scatp_hbm, u_hbm, c2_hbm, ch_hbm, uo_hbm,
               s0v, s1v, uv, c2v, chv, ov, sem):
    c = lax.axis_index("c")
    s = lax.axis_index("s")
    stripe = n_pad // (_NC * _NS)
    r0 = (c * _NS + s) * stripe
    pltpu.async_copy(scatp_hbm.at[0, pl.ds(r0, stripe)], s0v, sem)
    pltpu.async_copy(scatp_hbm.at[1, pl.ds(r0, stripe)], s1v, sem)
    pltpu.async_copy(u_hbm.at[pl.ds(r0, stripe)], uv, sem)
    pltpu.async_copy(c2_hbm.at[pl.ds(r0, stripe)], c2v, sem)
    pltpu.async_copy(ch_hbm.at[pl.ds(r0, stripe)], chv, sem)
    pltpu.make_async_copy(scatp_hbm.at[0, pl.ds(r0, stripe)], s0v, sem).wait()
    pltpu.make_async_copy(scatp_hbm.at[1, pl.ds(r0, stripe)], s1v, sem).wait()
    pltpu.make_async_copy(u_hbm.at[pl.ds(r0, stripe)], uv, sem).wait()
    pltpu.make_async_copy(c2_hbm.at[pl.ds(r0, stripe)], c2v, sem).wait()
    pltpu.make_async_copy(ch_hbm.at[pl.ds(r0, stripe)], chv, sem).wait()

    def rowb(r, carry):
        for k in range(d // 16):
            sl = (r, pl.ds(k * 16, 16))
            ov[sl] = c2v[sl] * (s0v[sl] + s1v[sl] + uv[sl]) + chv[sl]
        return carry

    lax.fori_loop(0, stripe, rowb, 0)
    pltpu.sync_copy(ov, uo_hbm.at[pl.ds(r0, stripe)])


def _make_combine(n_pad, d):
    stripe = n_pad // (_NC * _NS)
    buf = lambda: pltpu.VMEM((stripe, d), jnp.float32)
    scratch = [buf(), buf(), buf(), buf(), buf(), buf(),
               pltpu.SemaphoreType.DMA]
    return pl.kernel(
        functools.partial(_comb_body, n_pad, d),
        out_type=jax.ShapeDtypeStruct((n_pad, d), jnp.float32),
        mesh=_MESH,
        scratch_types=scratch,
        compiler_params=pltpu.CompilerParams(use_tc_tiling_on_sc=False),
    )


def _final_body(u_ref, sq_ref, o_ref):
    z = u_ref[...] * sq_ref[...]
    m = jnp.max(z, axis=1, keepdims=True)
    lse = jnp.log(jnp.sum(jnp.exp(z - m), axis=1, keepdims=True)) + m
    o_ref[...] = z - lse


def _tc_final(u, sq):
    n, d = u.shape
    blk = 2000
    spec = pl.BlockSpec((blk, d), lambda i: (i, 0))
    return pl.pallas_call(
        _final_body,
        grid=(n // blk,),
        in_specs=[spec] * 2,
        out_specs=spec,
        out_shape=jax.ShapeDtypeStruct((n, d), jnp.float32),
    )(u, sq)


def kernel(x, edge_index, W1, b1, W2, b2):
    n = x.shape[0]
    d = W2.shape[0]
    src = edge_index[0]
    dst = edge_index[1]

    e = src.shape[0]
    chunk = 125
    srcm = src.reshape(-1, chunk)
    dstm = dst.reshape(-1, chunk)

    n_pad = _rows_per_tile(n) * _NS
    degp = _sc_degree(dstm, n)
    u, chp, c2p, sq = _tc_prep(x, W1, b1, W2, b2,
                               degp[0, :n, 0], degp[1, :n, 0], n_pad)

    scat_fn = _make_scat(n, e, d, chunk)
    comb_fn = _make_combine(n_pad, d)
    zeros = jnp.zeros((n_pad, d), jnp.float32)
    for _ in range(_K):
        scatp = scat_fn(u, srcm, dstm, zeros)
        u = comb_fn(scatp, u, c2p, chp)
    return _tc_final(u[:n], sq)


# speedup vs baseline: 37.5860x; 1.0174x over previous
import functools

import jax
import jax.numpy as jnp
from jax import lax
from jax.experimental import pallas as pl
from jax.experimental.pallas import tpu as pltpu
from jax.experimental.pallas import tpu_sc as plsc

_ALPHA = 0.1
_K = 10
_CHUNK = 128

_NC = 2
_NS = 16
_MESH = plsc.VectorSubcoreMesh(core_axis_name="c", subcore_axis_name="s",
                               num_cores=_NC, num_subcores=_NS)


def _rows_per_tile(n):
    return -(-n // (_NS * 8)) * 8


def _deg_body(n, nch, ch, dstm_hbm, zeros_hbm, ones_hbm, out_hbm,
              acc, ones_v, dst_v, ssem):
    c = lax.axis_index("c")
    s = lax.axis_index("s")
    rows = _rows_per_tile(n)
    row0 = (c * _NS + s) * nch

    pltpu.sync_copy(zeros_hbm.at[pl.ds(s * rows, rows)],
                    acc.at[pl.ds(s * rows, rows)])
    pltpu.sync_copy(ones_hbm, ones_v)
    pltpu.sync_copy(dstm_hbm.at[pl.ds(row0, nch)], dst_v)
    plsc.subcore_barrier()

    def body(j, carry):
        @pl.when(j >= 2)
        def _():
            pltpu.make_async_copy(ones_v, acc.at[dst_v.at[j - 2]],
                                  ssem).wait()

        pltpu.async_copy(ones_v, acc.at[dst_v.at[j]], ssem, add=True)
        return carry

    lax.fori_loop(0, nch, body, 0)
    for t in range(max(nch - 2, 0), nch):
        pltpu.make_async_copy(ones_v, acc.at[dst_v.at[t]], ssem).wait()
    plsc.subcore_barrier()
    pltpu.sync_copy(acc.at[pl.ds(s * rows, rows)],
                    out_hbm.at[c, pl.ds(s * rows, rows)])


def _sc_degree(dstm, n):
    nrows, ch = dstm.shape
    nch = nrows // (_NC * _NS)
    n_pad = _rows_per_tile(n) * _NS
    zeros = jnp.zeros((n_pad, 8), jnp.float32)
    ones = jnp.ones((ch, 8), jnp.float32)
    scratch = [
        pltpu.VMEM_SHARED((n_pad, 8), jnp.float32),
        pltpu.VMEM((ch, 8), jnp.float32),
        pltpu.VMEM((nch, ch), jnp.int32),
        pltpu.SemaphoreType.DMA,
    ]
    kfn = pl.kernel(
        functools.partial(_deg_body, n, nch, ch),
        out_type=jax.ShapeDtypeStruct((_NC, n_pad, 8), jnp.float32),
        mesh=_MESH,
        scratch_types=scratch,
        compiler_params=pltpu.CompilerParams(use_tc_tiling_on_sc=False),
    )
    return kfn(dstm, zeros, ones)


def _scat_body(n, nch, d, ch, u_hbm, srcm_hbm, dstm_hbm, zeros_hbm, out_hbm,
               acc, src_v, dst_v, rbuf, gsem, ssem):
    c = lax.axis_index("c")
    s = lax.axis_index("s")
    rows = _rows_per_tile(n)
    row0 = (c * _NS + s) * nch

    pltpu.sync_copy(zeros_hbm.at[pl.ds(s * rows, rows)],
                    acc.at[pl.ds(s * rows, rows)])
    pltpu.sync_copy(srcm_hbm.at[pl.ds(row0, nch)], src_v)
    pltpu.sync_copy(dstm_hbm.at[pl.ds(row0, nch)], dst_v)
    pltpu.async_copy(u_hbm.at[src_v.at[0]], rbuf.at[0], gsem)
    plsc.subcore_barrier()

    if nch > 2:
        pltpu.async_copy(u_hbm.at[src_v.at[1]], rbuf.at[1], gsem)
        pltpu.async_copy(u_hbm.at[src_v.at[2]], rbuf.at[2], gsem)

    look = 3 if nch > 2 else 1

    def body(j, carry):
        @pl.when(j >= 8 - look)
        def _():
            pltpu.make_async_copy(rbuf.at[lax.rem(j - (8 - look), 8)],
                                  acc.at[dst_v.at[j - (8 - look)]],
                                  ssem).wait()

        @pl.when(j + look < nch)
        def _():
            pltpu.async_copy(u_hbm.at[src_v.at[j + look]],
                             rbuf.at[lax.rem(j + look, 8)], gsem)

        cur = lax.rem(j, 8)
        pltpu.make_async_copy(u_hbm.at[src_v.at[j]], rbuf.at[cur],
                              gsem).wait()
        pltpu.async_copy(rbuf.at[cur], acc.at[dst_v.at[j]], ssem, add=True)
        return carry

    lax.fori_loop(0, nch, body, 0)
    for t in range(max(nch - (8 - look), 0), nch):
        pltpu.make_async_copy(rbuf.at[t % 8], acc.at[dst_v.at[t]],
                              ssem).wait()
    plsc.subcore_barrier()
    pltpu.sync_copy(acc.at[pl.ds(s * rows, rows)],
                    out_hbm.at[c, pl.ds(s * rows, rows)])


def _make_scat(n, e, d, ch):
    assert e % (_NC * _NS * ch) == 0
    nch = e // (_NC * _NS * ch)
    n_pad = _rows_per_tile(n) * _NS
    scratch = [
        pltpu.VMEM_SHARED((n_pad, d), jnp.float32),
        pltpu.VMEM((nch, ch), jnp.int32),
        pltpu.VMEM((nch, ch), jnp.int32),
        pltpu.VMEM((8, ch, d), jnp.float32),
        pltpu.SemaphoreType.DMA,
        pltpu.SemaphoreType.DMA,
    ]
    return pl.kernel(
        functools.partial(_scat_body, n, nch, d, ch),
        out_type=jax.ShapeDtypeStruct((_NC, n_pad, d), jnp.float32),
        mesh=_MESH,
        scratch_types=scratch,
        compiler_params=pltpu.CompilerParams(use_tc_tiling_on_sc=False),
    )


def _prep_body(x_ref, w1_ref, b1_ref, w2_ref, b2_ref, d0_ref, d1_ref,
               u0_ref, ch_ref, c2_ref, sq_ref):
    x = x_ref[...]
    h1 = jax.nn.relu(
        lax.dot_general(x, w1_ref[...], (((1,), (1,)), ((), ())),
                        preferred_element_type=jnp.float32) + b1_ref[...])
    h2 = lax.dot_general(h1, w2_ref[...], (((1,), (1,)), ((), ())),
                         preferred_element_type=jnp.float32) + b2_ref[...]
    deg = d0_ref[...] + d1_ref[...] + 1.0
    dinv = lax.rsqrt(deg)
    u0 = h2 * dinv
    u0_ref[...] = u0
    ch_ref[...] = _ALPHA * u0
    c2_ref[...] = jnp.broadcast_to((1.0 - _ALPHA) * dinv * dinv, h2.shape)
    sq_ref[...] = jnp.broadcast_to(jnp.sqrt(deg), h2.shape)


def _tc_prep(x, w1, b1, w2, b2, d0, d1, n_pad):
    n, fin = x.shape
    d = w2.shape[0]
    blk = 2000
    grid = (n // blk,)
    f32 = jnp.float32
    out_shape = [jax.ShapeDtypeStruct((n_pad, d), f32)] * 3 + [
        jax.ShapeDtypeStruct((n, d), f32)]
    return pl.pallas_call(
        _prep_body,
        grid=grid,
        in_specs=[
            pl.BlockSpec((blk, fin), lambda i: (i, 0)),
            pl.BlockSpec(w1.shape, lambda i: (0, 0)),
            pl.BlockSpec((1, w1.shape[0]), lambda i: (0, 0)),
            pl.BlockSpec(w2.shape, lambda i: (0, 0)),
            pl.BlockSpec((1, w2.shape[0]), lambda i: (0, 0)),
            pl.BlockSpec((blk, 1), lambda i: (i, 0)),
            pl.BlockSpec((blk, 1), lambda i: (i, 0)),
        ],
        out_specs=[pl.BlockSpec((blk, d), lambda i: (i, 0))] * 4,
        out_shape=out_shape,
    )(x, w1, b1.reshape(1, -1), w2, b2.reshape(1, -1),
      d0.reshape(-1, 1), d1.reshape(-1, 1))


def _comb_body(n_pad, d, nb, scatp_hbm, u_hbm, c2_hbm, ch_hbm, uo_hbm,
               s0v, s1v, uv, c2v, chv, ov, sem, osem):
    c = lax.axis_index("c")
    s = lax.axis_index("s")
    stripe = n_pad // (_NC * _NS)
    br = stripe // nb
    r0 = (c * _NS + s) * stripe

    def pairs(b):
        sl = pl.ds(r0 + b * br, br)
        dl = pl.ds(b * br, br)
        return [(scatp_hbm.at[0, sl], s0v.at[dl]),
                (scatp_hbm.at[1, sl], s1v.at[dl]),
                (u_hbm.at[sl], uv.at[dl]),
                (c2_hbm.at[sl], c2v.at[dl]),
                (ch_hbm.at[sl], chv.at[dl])]

    for sp, dp in pairs(0):
        pltpu.async_copy(sp, dp, sem)
    for b in range(nb):
        if b + 1 < nb:
            for sp, dp in pairs(b + 1):
                pltpu.async_copy(sp, dp, sem)
        for sp, dp in pairs(b):
            pltpu.make_async_copy(sp, dp, sem).wait()

        def rowb(r, carry):
            for k in range(d // 16):
                sl = (r, pl.ds(k * 16, 16))
                ov[sl] = c2v[sl] * (s0v[sl] + s1v[sl] + uv[sl]) + chv[sl]
            return carry

        lax.fori_loop(b * br, (b + 1) * br, rowb, 0)
        pltpu.async_copy(ov.at[pl.ds(b * br, br)],
                         uo_hbm.at[pl.ds(r0 + b * br, br)], osem)
    for b in range(nb):
        pltpu.make_async_copy(ov.at[pl.ds(b * br, br)],
                              uo_hbm.at[pl.ds(r0 + b * br, br)], osem).wait()


def _make_combine(n_pad, d):
    stripe = n_pad // (_NC * _NS)
    nb = 4 if stripe % 4 == 0 else 1
    buf = lambda: pltpu.VMEM((stripe, d), jnp.float32)
    scratch = [buf(), buf(), buf(), buf(), buf(), buf(),
               pltpu.SemaphoreType.DMA, pltpu.SemaphoreType.DMA]
    return pl.kernel(
        functools.partial(_comb_body, n_pad, d, nb),
        out_type=jax.ShapeDtypeStruct((n_pad, d), jnp.float32),
        mesh=_MESH,
        scratch_types=scratch,
        compiler_params=pltpu.CompilerParams(use_tc_tiling_on_sc=False),
    )


def _final_body(u_ref, sq_ref, o_ref):
    z = u_ref[...] * sq_ref[...]
    m = jnp.max(z, axis=1, keepdims=True)
    lse = jnp.log(jnp.sum(jnp.exp(z - m), axis=1, keepdims=True)) + m
    o_ref[...] = z - lse


def _tc_final(u, sq):
    n, d = u.shape
    blk = 2000
    spec = pl.BlockSpec((blk, d), lambda i: (i, 0))
    return pl.pallas_call(
        _final_body,
        grid=(n // blk,),
        in_specs=[spec] * 2,
        out_specs=spec,
        out_shape=jax.ShapeDtypeStruct((n, d), jnp.float32),
    )(u, sq)


def kernel(x, edge_index, W1, b1, W2, b2):
    n = x.shape[0]
    d = W2.shape[0]
    src = edge_index[0]
    dst = edge_index[1]

    e = src.shape[0]
    chunk = 125
    srcm = src.reshape(-1, chunk)
    dstm = dst.reshape(-1, chunk)

    n_pad = _rows_per_tile(n) * _NS
    degp = _sc_degree(dstm, n)
    u, chp, c2p, sq = _tc_prep(x, W1, b1, W2, b2,
                               degp[0, :n, 0], degp[1, :n, 0], n_pad)

    scat_fn = _make_scat(n, e, d, chunk)
    comb_fn = _make_combine(n_pad, d)
    zeros = jnp.zeros((n_pad, d), jnp.float32)
    for _ in range(_K):
        scatp = scat_fn(u, srcm, dstm, zeros)
        u = comb_fn(scatp, u, c2p, chp)
    return _tc_final(u[:n], sq)
